# baseline (speedup 1.0000x reference)
import functools
from math import log

import jax
import jax.numpy as jnp
import numpy as np
from jax import lax
from jax.experimental import pallas as pl
from jax.experimental.pallas import tpu as pltpu
from jax.experimental.pallas import tpu_sc as plsc

_U32 = np.uint32
_F32 = jnp.float32
_TINY = np.float32(1.1754944e-38)

_KS0 = 0
_KS1 = 42
_KS2 = 0 ^ 42 ^ 0x1BD11BDA
_ROTS = (13, 15, 26, 6, 17, 29, 16, 24, 13, 15, 26, 6, 17, 29, 16, 24, 13, 15, 26, 6)
_INJ = ((_KS1, _KS2, 1), (_KS2, _KS0, 2), (_KS0, _KS1, 3), (_KS1, _KS2, 4), (_KS2, _KS0, 5))


def _threefry_bits(c1, c2):
    x0 = (c1 + _U32(_KS0)).astype(_U32)
    x1 = (c2 + _U32(_KS1)).astype(_U32)
    for i, r in enumerate(_ROTS):
        x0 = x0 + x1
        x1 = ((x1 << _U32(r)) | (x1 >> _U32(32 - r))) ^ x0
        if i % 4 == 3:
            a, b, c = _INJ[i // 4]
            x0 = x0 + _U32(a)
            x1 = x1 + _U32(b + c)
    return x0 ^ x1


def _log2(x):
    return jnp.log2(x)


def _escale_body(w_ref, e_ref):
    e_ref[...] = jnp.exp(-w_ref[...])


def _escale(weight, interpret=False):
    bsz, k_total = weight.shape
    return pl.pallas_call(
        _escale_body,
        out_shape=jax.ShapeDtypeStruct((bsz, k_total), _F32),
        interpret=interpret,
    )(weight)


def _sample_body(nb, chunk, n_total, k_total, e_ref, idx_ref, nw_ref):
    s = pl.program_id(0)
    logk = k_total.bit_length() - 1
    lognb = nb.bit_length() - 1
    su = s.astype(_U32)
    c1 = jnp.broadcast_to(su >> _U32(32 - lognb - logk), (nb, chunk)).astype(_U32)
    row_off = lax.broadcasted_iota(_U32, (nb, chunk), 0) << _U32(logk)
    lane = lax.broadcasted_iota(_U32, (nb, chunk), 1)
    c2base42 = (su << _U32(lognb + logk)) + row_off + lane + _U32(_KS1)
    kidx0 = lax.broadcasted_iota(jnp.int32, (nb, chunk), 1)
    nchunk = k_total // chunk
    jbits = max(nchunk.bit_length() - 1, 1)
    jmask = _U32(0xFFFFFFFF ^ ((1 << jbits) - 1))
    unroll = 4 if nchunk % 4 == 0 else 2

    def one(j):
        x1 = c2base42 + (j * chunk).astype(_U32)
        x0 = c1 + x1
        first = True
        for i, r in enumerate(_ROTS):
            if not first:
                x0 = x0 + x1
            first = False
            x1 = ((x1 << _U32(r)) | (x1 >> _U32(32 - r))) ^ x0
            if i % 4 == 3:
                a, b, c = _INJ[i // 4]
                x0 = x0 + _U32(a)
                x1 = x1 + _U32(b + c)
        bits = x0 ^ x1
        fb = (bits >> _U32(9)) | _U32(0x3F800000)
        u0 = lax.bitcast_convert_type(fb, _F32) - np.float32(1.0)
        lg = jnp.maximum(_log2(u0), np.float32(-126.0))
        p = lg * e_ref[0, 0, pl.ds(j * chunk, chunk)][None, :]
        pu = (lax.bitcast_convert_type(p, _U32) & jmask) | j.astype(_U32)
        return lax.bitcast_convert_type(pu, _F32)

    def step(i, carry):
        return tuple(
            jnp.maximum(m, one(unroll * i + u)) for u, m in enumerate(carry)
        )

    pinit = jnp.full((nb, chunk), -np.float32(3.0e38), _F32)
    ms = lax.fori_loop(0, nchunk // unroll, step, (pinit,) * unroll)
    pbest = ms[0]
    for m in ms[1:]:
        pbest = jnp.maximum(pbest, m)

    m = jnp.max(pbest, axis=1, keepdims=True)
    jdec = (lax.bitcast_convert_type(pbest, _U32) & _U32((1 << jbits) - 1))
    kbest = jdec.astype(jnp.int32) * chunk + kidx0
    ksel = jnp.where(pbest == m, kbest, jnp.int32(k_total))
    kmin = jnp.min(ksel, axis=1)
    b = s // (n_total // nb)
    idx_ref[0, 0, :] = kmin + b * k_total
    nw_ref[0, 0, :] = jnp.full((nb,), -log(n_total), _F32)


def _sample(weight, n_total, nb=8, chunk=512, interpret=False):
    bsz, k_total = weight.shape
    while chunk > 128 and (k_total // chunk) % 2:
        chunk //= 2
    steps = bsz * n_total // nb
    w3 = _escale(weight, interpret=interpret).reshape(bsz, 1, k_total)
    body = functools.partial(_sample_body, nb, chunk, n_total, k_total)
    idx3, nw3 = pl.pallas_call(
        body,
        grid=(steps,),
        in_specs=[pl.BlockSpec((1, 1, k_total), lambda s: (s // (n_total // nb), 0, 0))],
        out_specs=[
            pl.BlockSpec((1, 1, nb), lambda s: (s, 0, 0)),
            pl.BlockSpec((1, 1, nb), lambda s: (s, 0, 0)),
        ],
        out_shape=[
            jax.ShapeDtypeStruct((steps, 1, nb), jnp.int32),
            jax.ShapeDtypeStruct((steps, 1, nb), _F32),
        ],
        interpret=interpret,
    )(w3)
    return idx3.reshape(bsz, n_total), nw3.reshape(bsz, n_total)


_NW = 32
_CH = 128


def _gather(state2d, gidx):
    nw, nch, ch = gidx.shape
    rows_w = nch * ch
    total = nw * rows_w
    d = state2d.shape[1]

    mesh = plsc.VectorSubcoreMesh(core_axis_name="c", subcore_axis_name="s")

    @functools.partial(
        pl.kernel,
        mesh=mesh,
        out_type=jax.ShapeDtypeStruct((total, d), _F32),
        scratch_types=[
            pltpu.VMEM((nch, ch), jnp.int32),
            pltpu.VMEM((ch, d), _F32),
            pltpu.VMEM((ch, d), _F32),
            pltpu.SemaphoreType.DMA,
            pltpu.SemaphoreType.DMA,
        ],
        compiler_params=pltpu.CompilerParams(use_tc_tiling_on_sc=False),
    )
    def k(state_hbm, gidx_hbm, out_hbm, idx_v, buf0, buf1, sem0, sem1):
        wid = lax.axis_index("s") * 2 + lax.axis_index("c")
        base = wid * rows_w
        pltpu.sync_copy(gidx_hbm.at[wid], idx_v)
        bufs = (buf0, buf1)
        sems = (sem0, sem1)
        handles = [None, None]
        for c in range(nch):
            handles[c % 2] = pltpu.async_copy(
                state_hbm.at[idx_v.at[c]], bufs[c % 2], sems[c % 2])
            if c > 0:
                handles[(c - 1) % 2].wait()
                pltpu.sync_copy(bufs[(c - 1) % 2],
                                out_hbm.at[pl.ds(base + (c - 1) * ch, ch)])
        handles[(nch - 1) % 2].wait()
        pltpu.sync_copy(bufs[(nch - 1) % 2],
                        out_hbm.at[pl.ds(base + (nch - 1) * ch, ch)])

    return k(state2d, gidx)


def kernel(state, weight):
    bsz, n_total, d = state.shape
    k_total = weight.shape[1]
    gidx, new_weight = _sample(weight, n_total)
    gidx3 = gidx.reshape(_NW, (bsz * n_total) // (_NW * _CH), _CH)
    rows = _gather(state.reshape(bsz * k_total, d), gidx3)
    return rows.reshape(bsz, n_total, d), new_weight


# speedup vs baseline: 1.1041x; 1.1041x over previous
import functools
from math import log

import jax
import jax.numpy as jnp
import numpy as np
from jax import lax
from jax.experimental import pallas as pl
from jax.experimental.pallas import tpu as pltpu
from jax.experimental.pallas import tpu_sc as plsc

_U32 = np.uint32
_F32 = jnp.float32
_TINY = np.float32(1.1754944e-38)

_KS0 = 0
_KS1 = 42
_KS2 = 0 ^ 42 ^ 0x1BD11BDA
_ROTS = (13, 15, 26, 6, 17, 29, 16, 24, 13, 15, 26, 6, 17, 29, 16, 24, 13, 15, 26, 6)
_INJ = ((_KS1, _KS2, 1), (_KS2, _KS0, 2), (_KS0, _KS1, 3), (_KS1, _KS2, 4), (_KS2, _KS0, 5))


def _threefry_bits(c1, c2):
    x0 = (c1 + _U32(_KS0)).astype(_U32)
    x1 = (c2 + _U32(_KS1)).astype(_U32)
    for i, r in enumerate(_ROTS):
        x0 = x0 + x1
        x1 = ((x1 << _U32(r)) | (x1 >> _U32(32 - r))) ^ x0
        if i % 4 == 3:
            a, b, c = _INJ[i // 4]
            x0 = x0 + _U32(a)
            x1 = x1 + _U32(b + c)
    return x0 ^ x1


def _log2(x):
    return jnp.log2(x)


def _escale_body(w_ref, e_ref):
    e_ref[...] = jnp.exp(-w_ref[...])


def _escale(weight, interpret=False):
    bsz, k_total = weight.shape
    return pl.pallas_call(
        _escale_body,
        out_shape=jax.ShapeDtypeStruct((bsz, k_total), _F32),
        interpret=interpret,
    )(weight)


def _sample_body(nb, chunk, n_total, k_total, e_ref, idx_ref, nw_ref):
    s = pl.program_id(0)
    logk = k_total.bit_length() - 1
    lognb = nb.bit_length() - 1
    su = s.astype(_U32)
    c1 = jnp.broadcast_to(su >> _U32(32 - lognb - logk), (nb, chunk)).astype(_U32)
    row_off = lax.broadcasted_iota(_U32, (nb, chunk), 0) << _U32(logk)
    lane = lax.broadcasted_iota(_U32, (nb, chunk), 1)
    c2base42 = (su << _U32(lognb + logk)) + row_off + lane + _U32(_KS1)
    kidx0 = lax.broadcasted_iota(jnp.int32, (nb, chunk), 1)
    nchunk = k_total // chunk
    jbits = max(nchunk.bit_length() - 1, 1)
    jmask = _U32(0xFFFFFFFF ^ ((1 << jbits) - 1))
    unroll = 4 if nchunk % 4 == 0 else 2

    def one(j):
        x1 = c2base42 + (j * chunk).astype(_U32)
        x0 = c1 + x1
        first = True
        for i, r in enumerate(_ROTS):
            if not first:
                x0 = x0 + x1
            first = False
            x1 = ((x1 << _U32(r)) | (x1 >> _U32(32 - r))) ^ x0
            if i % 4 == 3:
                a, b, c = _INJ[i // 4]
                x0 = x0 + _U32(a)
                x1 = x1 + _U32(b + c)
        bits = x0 ^ x1
        fb = (bits >> _U32(9)) | _U32(0x3F800000)
        u0 = lax.bitcast_convert_type(fb, _F32) - np.float32(1.0)
        lg = jnp.maximum(_log2(u0), np.float32(-126.0))
        p = lg * e_ref[0, 0, pl.ds(j * chunk, chunk)][None, :]
        pu = (lax.bitcast_convert_type(p, _U32) & jmask) | j.astype(_U32)
        return lax.bitcast_convert_type(pu, _F32)

    def step(i, carry):
        return tuple(
            jnp.maximum(m, one(unroll * i + u)) for u, m in enumerate(carry)
        )

    pinit = jnp.full((nb, chunk), -np.float32(3.0e38), _F32)
    ms = lax.fori_loop(0, nchunk // unroll, step, (pinit,) * unroll)
    pbest = ms[0]
    for m in ms[1:]:
        pbest = jnp.maximum(pbest, m)

    m = jnp.max(pbest, axis=1, keepdims=True)
    jdec = (lax.bitcast_convert_type(pbest, _U32) & _U32((1 << jbits) - 1))
    kbest = (jdec << _U32(chunk.bit_length() - 1)).astype(jnp.int32) + kidx0
    ksel = jnp.where(pbest == m, kbest, jnp.int32(k_total))
    kmin = jnp.min(ksel, axis=1)
    b = s // (n_total // nb)
    idx_ref[0, 0, :] = kmin + b * k_total
    nw_ref[0, 0, :] = jnp.full((nb,), -log(n_total), _F32)


def _sample(weight, n_total, nb=16, chunk=512, interpret=False):
    bsz, k_total = weight.shape
    while chunk > 128 and (k_total // chunk) % 2:
        chunk //= 2
    steps = bsz * n_total // nb
    w3 = _escale(weight, interpret=interpret).reshape(bsz, 1, k_total)
    body = functools.partial(_sample_body, nb, chunk, n_total, k_total)
    idx3, nw3 = pl.pallas_call(
        body,
        grid=(steps,),
        in_specs=[pl.BlockSpec((1, 1, k_total), lambda s: (s // (n_total // nb), 0, 0))],
        out_specs=[
            pl.BlockSpec((1, 1, nb), lambda s: (s, 0, 0)),
            pl.BlockSpec((1, 1, nb), lambda s: (s, 0, 0)),
        ],
        out_shape=[
            jax.ShapeDtypeStruct((steps, 1, nb), jnp.int32),
            jax.ShapeDtypeStruct((steps, 1, nb), _F32),
        ],
        interpret=interpret,
    )(w3)
    return idx3.reshape(bsz, n_total), nw3.reshape(bsz, n_total)


_NW = 32
_CH = 128


def _gather(state2d, gidx):
    nw, nch, ch = gidx.shape
    rows_w = nch * ch
    total = nw * rows_w
    d = state2d.shape[1]

    mesh = plsc.VectorSubcoreMesh(core_axis_name="c", subcore_axis_name="s")

    @functools.partial(
        pl.kernel,
        mesh=mesh,
        out_type=jax.ShapeDtypeStruct((total, d), _F32),
        scratch_types=[
            pltpu.VMEM((nch, ch), jnp.int32),
            pltpu.VMEM((ch, d), _F32),
            pltpu.VMEM((ch, d), _F32),
            pltpu.SemaphoreType.DMA,
            pltpu.SemaphoreType.DMA,
        ],
        compiler_params=pltpu.CompilerParams(use_tc_tiling_on_sc=False),
    )
    def k(state_hbm, gidx_hbm, out_hbm, idx_v, buf0, buf1, sem0, sem1):
        wid = lax.axis_index("s") * 2 + lax.axis_index("c")
        base = wid * rows_w
        pltpu.sync_copy(gidx_hbm.at[wid], idx_v)
        bufs = (buf0, buf1)
        sems = (sem0, sem1)
        handles = [None, None]
        for c in range(nch):
            handles[c % 2] = pltpu.async_copy(
                state_hbm.at[idx_v.at[c]], bufs[c % 2], sems[c % 2])
            if c > 0:
                handles[(c - 1) % 2].wait()
                pltpu.sync_copy(bufs[(c - 1) % 2],
                                out_hbm.at[pl.ds(base + (c - 1) * ch, ch)])
        handles[(nch - 1) % 2].wait()
        pltpu.sync_copy(bufs[(nch - 1) % 2],
                        out_hbm.at[pl.ds(base + (nch - 1) * ch, ch)])

    return k(state2d, gidx)


def kernel(state, weight):
    bsz, n_total, d = state.shape
    k_total = weight.shape[1]
    gidx, new_weight = _sample(weight, n_total)
    gidx3 = gidx.reshape(_NW, (bsz * n_total) // (_NW * _CH), _CH)
    rows = _gather(state.reshape(bsz * k_total, d), gidx3)
    return rows.reshape(bsz, n_total, d), new_weight
